# baseline (speedup 1.0000x reference)
import jax
import jax.numpy as jnp
from jax.experimental import pallas as pl
from jax.experimental.pallas import tpu as pltpu

DIM = 128
K = 65536
BATCH = 4096
NCHUNK = (K - BATCH) // BATCH


def _body(ptr_ref, keys_ref, q_ref, out_ref, ptr_out_ref, tv, csem, wsem):
    p = jnp.clip(ptr_ref[0], 0, K - BATCH)
    p = pl.multiple_of(p, BATCH)

    copies = []
    for i in range(NCHUNK):
        base = i * BATCH
        col = jnp.where(base >= p, base + BATCH, base)
        col = pl.multiple_of(col, BATCH)
        c = pltpu.make_async_copy(
            q_ref.at[:, pl.ds(col, BATCH)], out_ref.at[:, pl.ds(col, BATCH)], csem
        )
        c.start()
        copies.append(c)

    def tr(i, carry):
        tv[:, pl.ds(i * DIM, DIM)] = keys_ref[pl.ds(i * DIM, DIM), :].T
        return carry

    jax.lax.fori_loop(0, BATCH // DIM, tr, 0)

    w = pltpu.make_async_copy(tv, out_ref.at[:, pl.ds(p, BATCH)], wsem)
    w.start()

    ptr_out_ref[0] = jax.lax.rem(ptr_ref[0] + BATCH, K)

    for c in copies:
        c.wait()
    w.wait()


def kernel(keys, queue, queue_ptr):
    ptr = queue_ptr.astype(jnp.int32)
    new_queue, new_ptr = pl.pallas_call(
        _body,
        grid=(),
        in_specs=[
            pl.BlockSpec(memory_space=pltpu.SMEM),
            pl.BlockSpec(memory_space=pltpu.VMEM),
            pl.BlockSpec(memory_space=pl.ANY),
        ],
        out_specs=[
            pl.BlockSpec(memory_space=pl.ANY),
            pl.BlockSpec(memory_space=pltpu.SMEM),
        ],
        out_shape=[
            jax.ShapeDtypeStruct((DIM, K), jnp.float32),
            jax.ShapeDtypeStruct((1,), jnp.int32),
        ],
        scratch_shapes=[
            pltpu.VMEM((DIM, BATCH), jnp.float32),
            pltpu.SemaphoreType.DMA,
            pltpu.SemaphoreType.DMA,
        ],
    )(ptr, keys, queue)
    return new_queue, new_ptr.astype(queue_ptr.dtype)


# speedup vs baseline: 31.3367x; 31.3367x over previous
import jax
import jax.numpy as jnp
from jax.experimental import pallas as pl
from jax.experimental.pallas import tpu as pltpu

DIM = 128
K = 65536
BATCH = 4096


def _update_body(ptr_ref, keys_ref, q_ref, out_ref, ptr_out_ref, tv, sem):
    p = ptr_ref[0]
    pc = jnp.clip(p, 0, K - BATCH)
    pc = pl.multiple_of(pc, DIM)

    def tr(i, carry):
        tv[:, pl.ds(i * DIM, DIM)] = keys_ref[pl.ds(i * DIM, DIM), :].T
        return carry

    jax.lax.fori_loop(0, BATCH // DIM, tr, 0)

    copy = pltpu.make_async_copy(tv, out_ref.at[:, pl.ds(pc, BATCH)], sem)
    copy.start()
    copy.wait()

    ptr_out_ref[0] = jax.lax.rem(p + BATCH, K)


def kernel(keys, queue, queue_ptr):
    ptr = queue_ptr.astype(jnp.int32)
    new_queue, new_ptr = pl.pallas_call(
        _update_body,
        grid=(),
        in_specs=[
            pl.BlockSpec(memory_space=pltpu.SMEM),
            pl.BlockSpec(memory_space=pltpu.VMEM),
            pl.BlockSpec(memory_space=pl.ANY),
        ],
        out_specs=[
            pl.BlockSpec(memory_space=pl.ANY),
            pl.BlockSpec(memory_space=pltpu.SMEM),
        ],
        out_shape=[
            jax.ShapeDtypeStruct((DIM, K), jnp.float32),
            jax.ShapeDtypeStruct((1,), jnp.int32),
        ],
        input_output_aliases={2: 0},
        scratch_shapes=[
            pltpu.VMEM((DIM, BATCH), jnp.float32),
            pltpu.SemaphoreType.DMA,
        ],
    )(ptr, keys, queue)
    return new_queue, new_ptr.astype(queue_ptr.dtype)


# speedup vs baseline: 35.4967x; 1.1328x over previous
import jax
import jax.numpy as jnp
from jax.experimental import pallas as pl
from jax.experimental.pallas import tpu as pltpu

DIM = 128
K = 65536
BATCH = 4096
CHUNK = 2048
NCH = (K - BATCH) // CHUNK
NBUF = 8
DEPTH = 4


def _body(ptr_ref, keys_ref, q_ref, out_ref, ptr_out_ref, bufs, tv, isem, osem, wsem):
    p = jnp.clip(ptr_ref[0], 0, K - BATCH)
    p = pl.multiple_of(p, BATCH)

    def col_of(i):
        base = i * CHUNK
        return pl.multiple_of(jnp.where(base >= p, base + BATCH, base), CHUNK)

    def start_in(i):
        b = i % NBUF
        c = pltpu.make_async_copy(
            q_ref.at[:, pl.ds(col_of(i), CHUNK)], bufs.at[b], isem.at[b]
        )
        c.start()
        return c

    def start_out(i):
        b = i % NBUF
        c = pltpu.make_async_copy(
            bufs.at[b], out_ref.at[:, pl.ds(col_of(i), CHUNK)], osem.at[b]
        )
        c.start()
        return c

    ins = {}
    outs = {}
    for i in range(DEPTH):
        ins[i] = start_in(i)

    def tr(i, carry):
        tv[:, pl.ds(i * DIM, DIM)] = keys_ref[pl.ds(i * DIM, DIM), :].T
        return carry

    jax.lax.fori_loop(0, BATCH // DIM, tr, 0)
    w = pltpu.make_async_copy(tv, out_ref.at[:, pl.ds(p, BATCH)], wsem)
    w.start()

    ptr_out_ref[0] = jax.lax.rem(ptr_ref[0] + BATCH, K)

    for i in range(NCH):
        ins[i].wait()
        outs[i] = start_out(i)
        j = i + DEPTH
        if j < NCH:
            if j - NBUF >= 0:
                outs[j - NBUF].wait()
            ins[j] = start_in(j)

    for i in range(max(0, NCH - NBUF), NCH):
        outs[i].wait()
    w.wait()


def kernel(keys, queue, queue_ptr):
    ptr = queue_ptr.astype(jnp.int32)
    new_queue, new_ptr = pl.pallas_call(
        _body,
        grid=(),
        in_specs=[
            pl.BlockSpec(memory_space=pltpu.SMEM),
            pl.BlockSpec(memory_space=pltpu.VMEM),
            pl.BlockSpec(memory_space=pl.ANY),
        ],
        out_specs=[
            pl.BlockSpec(memory_space=pl.ANY),
            pl.BlockSpec(memory_space=pltpu.SMEM),
        ],
        out_shape=[
            jax.ShapeDtypeStruct((DIM, K), jnp.float32),
            jax.ShapeDtypeStruct((1,), jnp.int32),
        ],
        scratch_shapes=[
            pltpu.VMEM((NBUF, DIM, CHUNK), jnp.float32),
            pltpu.VMEM((DIM, BATCH), jnp.float32),
            pltpu.SemaphoreType.DMA((NBUF,)),
            pltpu.SemaphoreType.DMA((NBUF,)),
            pltpu.SemaphoreType.DMA,
        ],
    )(ptr, keys, queue)
    return new_queue, new_ptr.astype(queue_ptr.dtype)


# speedup vs baseline: 38.0200x; 1.0711x over previous
import jax
import jax.numpy as jnp
from jax.experimental import pallas as pl
from jax.experimental.pallas import tpu as pltpu

DIM = 128
K = 65536
BATCH = 4096
CHUNK = 4096
NCH = (K - BATCH) // CHUNK
NBUF = 6
DEPTH = 3


def _body(ptr_ref, keys_ref, q_ref, out_ref, ptr_out_ref, bufs, tv, isem, osem, wsem):
    p = jnp.clip(ptr_ref[0], 0, K - BATCH)
    p = pl.multiple_of(p, BATCH)

    def col_of(i):
        base = i * CHUNK
        return pl.multiple_of(jnp.where(base >= p, base + BATCH, base), CHUNK)

    def start_in(i):
        b = i % NBUF
        c = pltpu.make_async_copy(
            q_ref.at[:, pl.ds(col_of(i), CHUNK)], bufs.at[b], isem.at[b]
        )
        c.start()
        return c

    def start_out(i):
        b = i % NBUF
        c = pltpu.make_async_copy(
            bufs.at[b], out_ref.at[:, pl.ds(col_of(i), CHUNK)], osem.at[b]
        )
        c.start()
        return c

    ins = {}
    outs = {}
    for i in range(DEPTH):
        ins[i] = start_in(i)

    def tr(i, carry):
        tv[:, pl.ds(i * DIM, DIM)] = keys_ref[pl.ds(i * DIM, DIM), :].T
        return carry

    jax.lax.fori_loop(0, BATCH // DIM, tr, 0)
    w = pltpu.make_async_copy(tv, out_ref.at[:, pl.ds(p, BATCH)], wsem)
    w.start()

    ptr_out_ref[0] = jax.lax.rem(ptr_ref[0] + BATCH, K)

    for i in range(NCH):
        ins[i].wait()
        outs[i] = start_out(i)
        j = i + DEPTH
        if j < NCH:
            if j - NBUF >= 0:
                outs[j - NBUF].wait()
            ins[j] = start_in(j)

    for i in range(max(0, NCH - NBUF), NCH):
        outs[i].wait()
    w.wait()


def kernel(keys, queue, queue_ptr):
    ptr = queue_ptr.astype(jnp.int32)
    new_queue, new_ptr = pl.pallas_call(
        _body,
        grid=(),
        in_specs=[
            pl.BlockSpec(memory_space=pltpu.SMEM),
            pl.BlockSpec(memory_space=pltpu.VMEM),
            pl.BlockSpec(memory_space=pl.ANY),
        ],
        out_specs=[
            pl.BlockSpec(memory_space=pl.ANY),
            pl.BlockSpec(memory_space=pltpu.SMEM),
        ],
        out_shape=[
            jax.ShapeDtypeStruct((DIM, K), jnp.float32),
            jax.ShapeDtypeStruct((1,), jnp.int32),
        ],
        scratch_shapes=[
            pltpu.VMEM((NBUF, DIM, CHUNK), jnp.float32),
            pltpu.VMEM((DIM, BATCH), jnp.float32),
            pltpu.SemaphoreType.DMA((NBUF,)),
            pltpu.SemaphoreType.DMA((NBUF,)),
            pltpu.SemaphoreType.DMA,
        ],
    )(ptr, keys, queue)
    return new_queue, new_ptr.astype(queue_ptr.dtype)


# speedup vs baseline: 40.6364x; 1.0688x over previous
import jax
import jax.numpy as jnp
from jax.experimental import pallas as pl
from jax.experimental.pallas import tpu as pltpu

DIM = 128
K = 65536
BATCH = 4096
CHUNK = 4096
NCH = (K - BATCH) // CHUNK
NBUF = 10
DEPTH = 5


def _body(ptr_ref, keys_ref, q_ref, out_ref, ptr_out_ref, bufs, tv, isem, osem, wsem):
    p = jnp.clip(ptr_ref[0], 0, K - BATCH)
    p = pl.multiple_of(p, BATCH)

    def col_of(i):
        base = i * CHUNK
        return pl.multiple_of(jnp.where(base >= p, base + BATCH, base), CHUNK)

    def start_in(i):
        b = i % NBUF
        c = pltpu.make_async_copy(
            q_ref.at[:, pl.ds(col_of(i), CHUNK)], bufs.at[b], isem.at[b]
        )
        c.start()
        return c

    def start_out(i):
        b = i % NBUF
        c = pltpu.make_async_copy(
            bufs.at[b], out_ref.at[:, pl.ds(col_of(i), CHUNK)], osem.at[b]
        )
        c.start()
        return c

    ins = {}
    outs = {}
    for i in range(DEPTH):
        ins[i] = start_in(i)

    def tr(i, carry):
        tv[:, pl.ds(i * DIM, DIM)] = keys_ref[pl.ds(i * DIM, DIM), :].T
        return carry

    jax.lax.fori_loop(0, BATCH // DIM, tr, 0)
    w = pltpu.make_async_copy(tv, out_ref.at[:, pl.ds(p, BATCH)], wsem)
    w.start()

    ptr_out_ref[0] = jax.lax.rem(ptr_ref[0] + BATCH, K)

    for i in range(NCH):
        ins[i].wait()
        outs[i] = start_out(i)
        j = i + DEPTH
        if j < NCH:
            if j - NBUF >= 0:
                outs[j - NBUF].wait()
            ins[j] = start_in(j)

    for i in range(max(0, NCH - NBUF), NCH):
        outs[i].wait()
    w.wait()


def kernel(keys, queue, queue_ptr):
    ptr = queue_ptr.astype(jnp.int32)
    new_queue, new_ptr = pl.pallas_call(
        _body,
        grid=(),
        in_specs=[
            pl.BlockSpec(memory_space=pltpu.SMEM),
            pl.BlockSpec(memory_space=pltpu.VMEM),
            pl.BlockSpec(memory_space=pl.ANY),
        ],
        out_specs=[
            pl.BlockSpec(memory_space=pl.ANY),
            pl.BlockSpec(memory_space=pltpu.SMEM),
        ],
        out_shape=[
            jax.ShapeDtypeStruct((DIM, K), jnp.float32),
            jax.ShapeDtypeStruct((1,), jnp.int32),
        ],
        scratch_shapes=[
            pltpu.VMEM((NBUF, DIM, CHUNK), jnp.float32),
            pltpu.VMEM((DIM, BATCH), jnp.float32),
            pltpu.SemaphoreType.DMA((NBUF,)),
            pltpu.SemaphoreType.DMA((NBUF,)),
            pltpu.SemaphoreType.DMA,
        ],
    )(ptr, keys, queue)
    return new_queue, new_ptr.astype(queue_ptr.dtype)


# speedup vs baseline: 41.3465x; 1.0175x over previous
import jax
import jax.numpy as jnp
from jax.experimental import pallas as pl
from jax.experimental.pallas import tpu as pltpu

DIM = 128
K = 65536
BATCH = 4096
CHUNK = 4096
NCH = (K - BATCH) // CHUNK
NBUF = 15
DEPTH = 15


def _body(ptr_ref, keys_ref, q_ref, out_ref, ptr_out_ref, bufs, tv, isem, osem, wsem):
    p = jnp.clip(ptr_ref[0], 0, K - BATCH)
    p = pl.multiple_of(p, BATCH)

    def col_of(i):
        base = i * CHUNK
        return pl.multiple_of(jnp.where(base >= p, base + BATCH, base), CHUNK)

    def start_in(i):
        b = i % NBUF
        c = pltpu.make_async_copy(
            q_ref.at[:, pl.ds(col_of(i), CHUNK)], bufs.at[b], isem.at[b]
        )
        c.start()
        return c

    def start_out(i):
        b = i % NBUF
        c = pltpu.make_async_copy(
            bufs.at[b], out_ref.at[:, pl.ds(col_of(i), CHUNK)], osem.at[b]
        )
        c.start()
        return c

    ins = {}
    outs = {}
    for i in range(DEPTH):
        ins[i] = start_in(i)

    def tr(i, carry):
        tv[:, pl.ds(i * DIM, DIM)] = keys_ref[pl.ds(i * DIM, DIM), :].T
        return carry

    jax.lax.fori_loop(0, BATCH // DIM, tr, 0)
    w = pltpu.make_async_copy(tv, out_ref.at[:, pl.ds(p, BATCH)], wsem)
    w.start()

    ptr_out_ref[0] = jax.lax.rem(ptr_ref[0] + BATCH, K)

    for i in range(NCH):
        ins[i].wait()
        outs[i] = start_out(i)
        j = i + DEPTH
        if j < NCH:
            if j - NBUF >= 0:
                outs[j - NBUF].wait()
            ins[j] = start_in(j)

    for i in range(max(0, NCH - NBUF), NCH):
        outs[i].wait()
    w.wait()


def kernel(keys, queue, queue_ptr):
    ptr = queue_ptr.astype(jnp.int32)
    new_queue, new_ptr = pl.pallas_call(
        _body,
        grid=(),
        in_specs=[
            pl.BlockSpec(memory_space=pltpu.SMEM),
            pl.BlockSpec(memory_space=pltpu.VMEM),
            pl.BlockSpec(memory_space=pl.ANY),
        ],
        out_specs=[
            pl.BlockSpec(memory_space=pl.ANY),
            pl.BlockSpec(memory_space=pltpu.SMEM),
        ],
        out_shape=[
            jax.ShapeDtypeStruct((DIM, K), jnp.float32),
            jax.ShapeDtypeStruct((1,), jnp.int32),
        ],
        scratch_shapes=[
            pltpu.VMEM((NBUF, DIM, CHUNK), jnp.float32),
            pltpu.VMEM((DIM, BATCH), jnp.float32),
            pltpu.SemaphoreType.DMA((NBUF,)),
            pltpu.SemaphoreType.DMA((NBUF,)),
            pltpu.SemaphoreType.DMA,
        ],
    )(ptr, keys, queue)
    return new_queue, new_ptr.astype(queue_ptr.dtype)
